# baseline (speedup 1.0000x reference)
import functools

import jax
import jax.numpy as jnp
from jax import lax
from jax.experimental import pallas as pl
from jax.experimental.pallas import tpu as pltpu
from jax.experimental.pallas import tpu_sc as plsc

EMBED_DIM = 128
NUM_CORES = 2
NUM_SUBCORES = 16
NUM_WORKERS = NUM_CORES * NUM_SUBCORES

CHUNK_ROWS = 128
IDX_BLOCK = 32


def _embed_body(x2d_hbm, table_hbm, out_hbm, idx_v, rows_v, sem):
  n_rows = out_hbm.shape[0]
  rows_per_worker = n_rows // NUM_WORKERS
  chunks_per_worker = rows_per_worker // CHUNK_ROWS
  n_idx_blocks = chunks_per_worker // IDX_BLOCK

  wid = lax.axis_index("s") * NUM_CORES + lax.axis_index("c")
  chunk0 = wid * chunks_per_worker

  def outer(i, _):
    iblk = pl.multiple_of(chunk0 + i * IDX_BLOCK, IDX_BLOCK)
    pltpu.sync_copy(x2d_hbm.at[pl.ds(iblk, IDX_BLOCK)], idx_v)

    def inner(j, _):
      c = pl.multiple_of(chunk0 + i * IDX_BLOCK + j, 1)
      row0 = pl.multiple_of(c * CHUNK_ROWS, CHUNK_ROWS)
      pltpu.async_copy(table_hbm.at[idx_v.at[j]], rows_v, sem).wait()
      pltpu.sync_copy(rows_v, out_hbm.at[pl.ds(row0, CHUNK_ROWS)])
      return ()

    lax.fori_loop(0, IDX_BLOCK, inner, (), unroll=False)
    return ()

  lax.fori_loop(0, n_idx_blocks, outer, (), unroll=False)


@jax.jit
def kernel(x, table):
  batch, seq = x.shape
  n_rows = batch * seq
  x2d = x.reshape(n_rows // EMBED_DIM, EMBED_DIM).astype(jnp.int32)
  table = table.astype(jnp.float32)

  mesh = plsc.VectorSubcoreMesh(core_axis_name="c", subcore_axis_name="s")
  run = pl.kernel(
      _embed_body,
      out_type=jax.ShapeDtypeStruct((n_rows, EMBED_DIM), jnp.float32),
      mesh=mesh,
      scratch_types=[
          pltpu.VMEM((IDX_BLOCK, EMBED_DIM), jnp.int32),
          pltpu.VMEM((CHUNK_ROWS, EMBED_DIM), jnp.float32),
          pltpu.SemaphoreType.DMA,
      ],
  )
  out = run(x2d, table)
  return out.reshape(batch, seq, EMBED_DIM)
